# baseline (speedup 1.0000x reference)
import functools

import jax
import jax.numpy as jnp
import numpy as np
from jax import lax
from jax.experimental import pallas as pl
from jax.experimental.pallas import tpu as pltpu
from jax.experimental.pallas import tpu_sc as plsc

DIM = 32
NUM_CORES = 2
NUM_SUBCORES = 16
NW = NUM_CORES * NUM_SUBCORES
EPS = 1e-4


@functools.lru_cache(maxsize=None)
def _build(nb, nt):
    cb = 8
    rpc = cb * nt
    half = rpc // 2
    assert nb % (NW * cb) == 0 and half % 8 == 0
    bpw_b = nb // NW
    nch = bpw_b // cb
    mesh = plsc.VectorSubcoreMesh(core_axis_name="c", subcore_axis_name="s")

    @functools.partial(
        pl.kernel,
        out_type=jax.ShapeDtypeStruct((nb, nt, DIM), jnp.float32),
        mesh=mesh,
        scratch_types=[
            pltpu.VMEM((2, half), jnp.int32),
            pltpu.VMEM((rpc, DIM), jnp.float32),
            pltpu.SemaphoreType.DMA,
        ],
        compiler_params=pltpu.CompilerParams(
            needs_layout_passes=False, use_tc_tiling_on_sc=False),
    )
    def impl(idx_hbm, table_hbm, out_hbm, idx_v, rows_v, sem):
        wid = lax.axis_index("s") * NUM_CORES + lax.axis_index("c")
        b_base = wid * bpw_b

        def chunk_body(ci, carry):
            b0 = b_base + ci * cb
            off = b0 * nt
            pltpu.sync_copy(idx_hbm.at[pl.ds(off, half)], idx_v.at[0])
            pltpu.sync_copy(idx_hbm.at[pl.ds(off + half, half)], idx_v.at[1])
            c0 = pltpu.async_copy(
                table_hbm.at[idx_v.at[0]], rows_v.at[pl.ds(0, half)], sem)
            c1 = pltpu.async_copy(
                table_hbm.at[idx_v.at[1]], rows_v.at[pl.ds(half, half)], sem)
            c0.wait()
            c1.wait()
            for k in range(cb):
                pltpu.sync_copy(rows_v.at[pl.ds(k * nt, nt)], out_hbm.at[b0 + k])
            return carry

        lax.fori_loop(0, nch, chunk_body, 0)

    return impl


def kernel(x, weight):
    nb, nt = x.shape
    xf = jnp.reshape(x, (nb * nt,)).astype(jnp.int32)
    norm = jnp.sqrt(jnp.sum(weight * weight, axis=1, keepdims=True))
    wn = weight / (EPS + norm * np.float32(np.sqrt(1.0 / DIM)))
    return _build(nb, nt)(xf, wn)
